# baseline (speedup 1.0000x reference)
import jax
import jax.numpy as jnp
from jax import lax
from jax.experimental import pallas as pl
from jax.experimental.pallas import tpu as pltpu
from jax.experimental.pallas import tpu_sc as plsc

LATENT = 32
BATCH = 16384

NUM_CORES = 2
NUM_SUBCORES = 16
LANES = 16
NW = NUM_CORES * NUM_SUBCORES
B_PER_W = BATCH // NW
IDX_BLKS = B_PER_W // 128
CHUNKS = B_PER_W // LANES


def _cf_body(binp_hbm, user_hbm, item_hbm, out_hbm,
             binp_v, idxu_v, idxi_v, urows_v, irows_v, out_v, sem):
    wid = lax.axis_index("s") * NUM_CORES + lax.axis_index("c")
    base = wid * B_PER_W

    pltpu.sync_copy(binp_hbm.at[pl.ds(base, B_PER_W), :], binp_v)

    lane = lax.iota(jnp.int32, LANES)
    col_user = jnp.zeros((LANES,), jnp.int32)
    col_item = jnp.full((LANES,), 2, jnp.int32)

    for blk in range(IDX_BLKS):
        for c in range(8):
            rows = (blk * 8 + c) * LANES + lane
            uf = plsc.load_gather(binp_v, [rows, col_user])
            itf = plsc.load_gather(binp_v, [rows, col_item])
            idxu_v[blk, pl.ds(c * LANES, LANES)] = uf.astype(jnp.int32)
            idxi_v[blk, pl.ds(c * LANES, LANES)] = itf.astype(jnp.int32)

    copies = []
    for blk in range(IDX_BLKS):
        copies.append(pltpu.async_copy(
            user_hbm.at[idxu_v.at[blk]],
            urows_v.at[pl.ds(blk * 128, 128), :], sem))
        copies.append(pltpu.async_copy(
            item_hbm.at[idxi_v.at[blk]],
            irows_v.at[pl.ds(blk * 128, 128), :], sem))
    for cp in copies:
        cp.wait()

    def chunk_body(c, carry):
        rows = c * LANES + lane
        acc = jnp.zeros((LANES,), jnp.float32)
        for d in range(LATENT):
            col = jnp.full((LANES,), d, jnp.int32)
            uv = plsc.load_gather(urows_v, [rows, col])
            iv = plsc.load_gather(irows_v, [rows, col])
            acc = acc + uv * iv
        acc = jnp.clip(acc, 0.0, 5.0)
        out_v[pl.ds(c * LANES, LANES)] = acc
        return carry

    lax.fori_loop(0, CHUNKS, chunk_body, 0)

    pltpu.sync_copy(out_v, out_hbm.at[pl.ds(base, B_PER_W)])


@jax.jit
def kernel(batched_inputs, user_hidden_emb, item_hidden_emb):
    mesh = plsc.VectorSubcoreMesh(core_axis_name="c", subcore_axis_name="s")
    run = pl.kernel(
        _cf_body,
        out_type=jax.ShapeDtypeStruct((BATCH,), jnp.float32),
        mesh=mesh,
        scratch_types=[
            pltpu.VMEM((B_PER_W, 4), jnp.float32),
            pltpu.VMEM((IDX_BLKS, 128), jnp.int32),
            pltpu.VMEM((IDX_BLKS, 128), jnp.int32),
            pltpu.VMEM((B_PER_W, LATENT), jnp.float32),
            pltpu.VMEM((B_PER_W, LATENT), jnp.float32),
            pltpu.VMEM((B_PER_W,), jnp.float32),
            pltpu.SemaphoreType.DMA,
        ],
        compiler_params=pltpu.CompilerParams(
            needs_layout_passes=False, use_tc_tiling_on_sc=False),
    )
    return run(batched_inputs, user_hidden_emb, item_hidden_emb)


# speedup vs baseline: 4.0076x; 4.0076x over previous
import jax
import jax.numpy as jnp
from jax import lax
from jax.experimental import pallas as pl
from jax.experimental.pallas import tpu as pltpu
from jax.experimental.pallas import tpu_sc as plsc

LATENT = 32
BATCH = 16384

NUM_CORES = 2
NUM_SUBCORES = 16
LANES = 16
NW = NUM_CORES * NUM_SUBCORES
B_PER_W = BATCH // NW
CHUNKS = B_PER_W // LANES
NBUF = 8


def _cf_body(idxu_hbm, idxi_hbm, user_t_hbm, item_t_hbm, out_hbm,
             idxu_v, idxi_v, ublk_v, vblk_v, out_v, usems, vsems):
    wid = lax.axis_index("s") * NUM_CORES + lax.axis_index("c")
    base = wid * B_PER_W

    pltpu.sync_copy(idxu_hbm.at[pl.ds(base, B_PER_W)], idxu_v)
    pltpu.sync_copy(idxi_hbm.at[pl.ds(base, B_PER_W)], idxi_v)

    lane = lax.iota(jnp.int32, LANES)
    rows_lo = lane
    rows_hi = lane + LANES

    def fire(slot, cu, ci):
        off_u = pl.multiple_of(cu * 128, 128)
        off_i = pl.multiple_of(ci * 128, 128)
        pltpu.async_copy(
            user_t_hbm.at[:, pl.ds(off_u, 128)], ublk_v.at[slot],
            usems.at[slot])
        pltpu.async_copy(
            item_t_hbm.at[:, pl.ds(off_i, 128)], vblk_v.at[slot],
            vsems.at[slot])

    def drain(slot):
        pltpu.make_async_copy(
            user_t_hbm.at[:, pl.ds(0, 128)], ublk_v.at[slot],
            usems.at[slot]).wait()
        pltpu.make_async_copy(
            item_t_hbm.at[:, pl.ds(0, 128)], vblk_v.at[slot],
            vsems.at[slot]).wait()

    u0 = idxu_v[pl.ds(0, LANES)]
    i0 = idxi_v[pl.ds(0, LANES)]
    cu0 = lax.shift_right_logical(u0, 7)
    ci0 = lax.shift_right_logical(i0, 7)
    for k in range(NBUF):
        fire(k, cu0[k], ci0[k])

    def step(c, carry):
        uvec = idxu_v[pl.ds(c * LANES, LANES)]
        ivec = idxi_v[pl.ds(c * LANES, LANES)]
        cu_vec = lax.shift_right_logical(uvec, 7)
        ci_vec = lax.shift_right_logical(ivec, 7)
        lu_vec = lax.bitwise_and(uvec, 127)
        li_vec = lax.bitwise_and(ivec, 127)
        cn = jnp.where(c + 1 < CHUNKS, c + 1, 0)
        nuvec = idxu_v[pl.ds(cn * LANES, LANES)]
        nivec = idxi_v[pl.ds(cn * LANES, LANES)]
        pu_vec = lax.shift_right_logical(nuvec, 7)
        pi_vec = lax.shift_right_logical(nivec, 7)

        acc = jnp.zeros((LANES,), jnp.float32)
        for k in range(LANES):
            slot = k % NBUF
            drain(slot)
            slot_v = jnp.full((LANES,), slot, jnp.int32)
            lu = jnp.full((LANES,), lu_vec[k], jnp.int32)
            li = jnp.full((LANES,), li_vec[k], jnp.int32)
            u_lo = plsc.load_gather(ublk_v, [slot_v, rows_lo, lu])
            u_hi = plsc.load_gather(ublk_v, [slot_v, rows_hi, lu])
            v_lo = plsc.load_gather(vblk_v, [slot_v, rows_lo, li])
            v_hi = plsc.load_gather(vblk_v, [slot_v, rows_hi, li])
            p = u_lo * v_lo + u_hi * v_hi
            s = jnp.sum(p)
            acc = jnp.where(lane == k, s, acc)
            if k < NBUF:
                fire(slot, cu_vec[k + NBUF], ci_vec[k + NBUF])
            else:
                @pl.when(c + 1 < CHUNKS)
                def _():
                    fire(slot, pu_vec[k - NBUF], pi_vec[k - NBUF])
        acc = jnp.clip(acc, 0.0, 5.0)
        out_v[pl.ds(c * LANES, LANES)] = acc
        return carry

    lax.fori_loop(0, CHUNKS, step, 0)

    pltpu.sync_copy(out_v, out_hbm.at[pl.ds(base, B_PER_W)])


@jax.jit
def kernel(batched_inputs, user_hidden_emb, item_hidden_emb):
    idx_user = batched_inputs[:, 0].astype(jnp.int32)
    idx_item = batched_inputs[:, 2].astype(jnp.int32)
    mesh = plsc.VectorSubcoreMesh(core_axis_name="c", subcore_axis_name="s")
    run = pl.kernel(
        _cf_body,
        out_type=jax.ShapeDtypeStruct((BATCH,), jnp.float32),
        mesh=mesh,
        scratch_types=[
            pltpu.VMEM((B_PER_W,), jnp.int32),
            pltpu.VMEM((B_PER_W,), jnp.int32),
            pltpu.VMEM((NBUF, LATENT, 128), jnp.float32),
            pltpu.VMEM((NBUF, LATENT, 128), jnp.float32),
            pltpu.VMEM((B_PER_W,), jnp.float32),
            pltpu.SemaphoreType.DMA((NBUF,)),
            pltpu.SemaphoreType.DMA((NBUF,)),
        ],
        compiler_params=pltpu.CompilerParams(
            needs_layout_passes=False, use_tc_tiling_on_sc=True),
    )
    return run(idx_user, idx_item, user_hidden_emb.T, item_hidden_emb.T)
